# baseline (speedup 1.0000x reference)
import functools

import jax
import jax.numpy as jnp
from jax import lax
from jax.experimental import pallas as pl
from jax.experimental.pallas import tpu as pltpu
from jax.experimental.pallas import tpu_sc as plsc

NUM_INPUTS = 20
BATCH = 16384
LANES = 16
NUM_WORKERS = 32
B_PER_W = BATCH // NUM_WORKERS
GROUPS = B_PER_W // LANES
GCHUNK = 128
NCHUNK = B_PER_W // GCHUNK

_LOGIT_LO = -16.118095
_LOGIT_HI = 15.942385


def _logic_unit_body(x_hbm, lut_hbm, out_hbm, probs_hbm, logits_hbm,
                     x_v, idx_v, vals_v, out_v, probs_v, logits_v, sem):
  wid = lax.axis_index("s") * 2 + lax.axis_index("c")
  base = wid * B_PER_W

  pltpu.sync_copy(x_hbm.at[pl.ds(base * NUM_INPUTS, B_PER_W * NUM_INPUTS)],
                  x_v)

  def pack_group(g, carry):
    flat0 = g * (LANES * NUM_INPUTS) + lax.iota(jnp.int32, LANES) * NUM_INPUTS
    acc = jnp.zeros((LANES,), jnp.float32)
    for k in range(NUM_INPUTS):
      bit = plsc.load_gather(x_v, [flat0 + k])
      acc = acc * 2.0 + bit
    idx_v[pl.ds(g * LANES, LANES)] = acc.astype(jnp.int32)
    return carry

  lax.fori_loop(0, GROUPS, pack_group, 0)

  copies = []
  for j in range(NCHUNK):
    copies.append(pltpu.async_copy(
        lut_hbm.at[idx_v.at[pl.ds(j * GCHUNK, GCHUNK)]],
        vals_v.at[pl.ds(j * GCHUNK, GCHUNK)], sem))
  for c in copies:
    c.wait()

  for v in range(GROUPS):
    off = v * LANES
    g = vals_v[pl.ds(off, LANES)]
    p = 1.0 / (1.0 + jnp.exp(-g))
    out_v[pl.ds(off, LANES)] = jnp.where(
        p >= 0.5, jnp.float32(1.0), jnp.float32(0.0))
    probs_v[pl.ds(off, LANES)] = p
    logits_v[pl.ds(off, LANES)] = 5.0 * jnp.clip(g, _LOGIT_LO, _LOGIT_HI)

  pltpu.sync_copy(out_v, out_hbm.at[pl.ds(base, B_PER_W)])
  pltpu.sync_copy(probs_v, probs_hbm.at[pl.ds(base, B_PER_W)])
  pltpu.sync_copy(logits_v, logits_hbm.at[pl.ds(base, B_PER_W)])


_OUT = jax.ShapeDtypeStruct((BATCH,), jnp.float32)

_logic_unit_sc = functools.partial(
    pl.kernel,
    out_type=(_OUT, _OUT, _OUT),
    mesh=plsc.VectorSubcoreMesh(core_axis_name="c", subcore_axis_name="s"),
    compiler_params=pltpu.CompilerParams(needs_layout_passes=False),
    scratch_types=[
        pltpu.VMEM((B_PER_W * NUM_INPUTS,), jnp.float32),
        pltpu.VMEM((B_PER_W,), jnp.int32),
        pltpu.VMEM((B_PER_W,), jnp.float32),
        pltpu.VMEM((B_PER_W,), jnp.float32),
        pltpu.VMEM((B_PER_W,), jnp.float32),
        pltpu.VMEM((B_PER_W,), jnp.float32),
        pltpu.SemaphoreType.DMA,
    ],
)(_logic_unit_body)


@jax.jit
def kernel(x, lut_params):
  return _logic_unit_sc(x.reshape(-1), lut_params)


# speedup vs baseline: 1.0149x; 1.0149x over previous
import functools

import jax
import jax.numpy as jnp
from jax import lax
from jax.experimental import pallas as pl
from jax.experimental.pallas import tpu as pltpu
from jax.experimental.pallas import tpu_sc as plsc

NUM_INPUTS = 20
BATCH = 16384
LANES = 16
NUM_WORKERS = 32
B_PER_W = BATCH // NUM_WORKERS
GCHUNK = 128
NCHUNK = B_PER_W // GCHUNK
GROUPS_PER_CHUNK = GCHUNK // LANES

_W = [float(2 ** (NUM_INPUTS - 1 - k)) for k in range(NUM_INPUTS)]

_LOGIT_LO = -16.118095
_LOGIT_HI = 15.942385


def _tree_sum(vals):
  while len(vals) > 1:
    nxt = [a + b for a, b in zip(vals[::2], vals[1::2])]
    if len(vals) % 2:
      nxt.append(vals[-1])
    vals = nxt
  return vals[0]


def _logic_unit_body(x_hbm, lut_hbm, out_hbm, probs_hbm, logits_hbm,
                     x_v, idx_v, vals_v, out_v, probs_v, logits_v,
                     semx, semg, semo):
  wid = lax.axis_index("s") * 2 + lax.axis_index("c")
  base = wid * B_PER_W

  xcopies = []
  for j in range(NCHUNK):
    off = j * GCHUNK * NUM_INPUTS
    xcopies.append(pltpu.async_copy(
        x_hbm.at[pl.ds(base * NUM_INPUTS + off, GCHUNK * NUM_INPUTS)],
        x_v.at[pl.ds(off, GCHUNK * NUM_INPUTS)], semx.at[j]))

  gcopies = []
  for j in range(NCHUNK):
    xcopies[j].wait()
    for g in range(GROUPS_PER_CHUNK):
      row0 = j * GCHUNK + g * LANES
      flat0 = row0 * NUM_INPUTS + lax.iota(jnp.int32, LANES) * NUM_INPUTS
      bits = [plsc.load_gather(x_v, [flat0 + k]) for k in range(NUM_INPUTS)]
      acc = _tree_sum([b * _W[k] for k, b in enumerate(bits)])
      idx_v[pl.ds(row0, LANES)] = acc.astype(jnp.int32)
    gcopies.append(pltpu.async_copy(
        lut_hbm.at[idx_v.at[pl.ds(j * GCHUNK, GCHUNK)]],
        vals_v.at[pl.ds(j * GCHUNK, GCHUNK)], semg.at[j]))

  ocopies = []
  for j in range(NCHUNK):
    gcopies[j].wait()
    for g in range(GROUPS_PER_CHUNK):
      off = j * GCHUNK + g * LANES
      gval = vals_v[pl.ds(off, LANES)]
      p = 1.0 / (1.0 + jnp.exp(-gval))
      out_v[pl.ds(off, LANES)] = jnp.where(
          p >= 0.5, jnp.float32(1.0), jnp.float32(0.0))
      probs_v[pl.ds(off, LANES)] = p
      logits_v[pl.ds(off, LANES)] = 5.0 * jnp.clip(gval, _LOGIT_LO, _LOGIT_HI)
    src = pl.ds(j * GCHUNK, GCHUNK)
    dst = pl.ds(base + j * GCHUNK, GCHUNK)
    ocopies.append(pltpu.async_copy(out_v.at[src], out_hbm.at[dst],
                                    semo.at[3 * j]))
    ocopies.append(pltpu.async_copy(probs_v.at[src], probs_hbm.at[dst],
                                    semo.at[3 * j + 1]))
    ocopies.append(pltpu.async_copy(logits_v.at[src], logits_hbm.at[dst],
                                    semo.at[3 * j + 2]))
  for c in ocopies:
    c.wait()


_OUT = jax.ShapeDtypeStruct((BATCH,), jnp.float32)

_logic_unit_sc = functools.partial(
    pl.kernel,
    out_type=(_OUT, _OUT, _OUT),
    mesh=plsc.VectorSubcoreMesh(core_axis_name="c", subcore_axis_name="s"),
    compiler_params=pltpu.CompilerParams(needs_layout_passes=False),
    scratch_types=[
        pltpu.VMEM((B_PER_W * NUM_INPUTS,), jnp.float32),
        pltpu.VMEM((B_PER_W,), jnp.int32),
        pltpu.VMEM((B_PER_W,), jnp.float32),
        pltpu.VMEM((B_PER_W,), jnp.float32),
        pltpu.VMEM((B_PER_W,), jnp.float32),
        pltpu.VMEM((B_PER_W,), jnp.float32),
        pltpu.SemaphoreType.DMA((NCHUNK,)),
        pltpu.SemaphoreType.DMA((NCHUNK,)),
        pltpu.SemaphoreType.DMA((3 * NCHUNK,)),
    ],
)(_logic_unit_body)


@jax.jit
def kernel(x, lut_params):
  return _logic_unit_sc(x.reshape(-1), lut_params)


# speedup vs baseline: 1.1300x; 1.1133x over previous
import functools

import jax
import jax.numpy as jnp
from jax import lax
from jax.experimental import pallas as pl
from jax.experimental.pallas import tpu as pltpu
from jax.experimental.pallas import tpu_sc as plsc

NUM_INPUTS = 20
BATCH = 16384
LANES = 16
NUM_WORKERS = 32
B_PER_W = BATCH // NUM_WORKERS
GCHUNK = 128
NCHUNK = B_PER_W // GCHUNK
GROUPS_PER_CHUNK = GCHUNK // LANES

_W = [float(2 ** (NUM_INPUTS - 1 - k)) for k in range(NUM_INPUTS)]

_LOGIT_LO = -16.118095
_LOGIT_HI = 15.942385


def _tree_sum(vals):
  while len(vals) > 1:
    nxt = [a + b for a, b in zip(vals[::2], vals[1::2])]
    if len(vals) % 2:
      nxt.append(vals[-1])
    vals = nxt
  return vals[0]


def _logic_unit_body(x_hbm, lut_hbm, out_hbm, probs_hbm, logits_hbm,
                     x_v, idx_v, vals_v, out_v, probs_v, logits_v,
                     semx, semg, semo):
  wid = lax.axis_index("s") * 2 + lax.axis_index("c")
  base = wid * B_PER_W

  xcopies = []
  for j in range(NCHUNK):
    off = j * GCHUNK
    xcopies.append(pltpu.async_copy(
        x_hbm.at[pl.ds(base + off, GCHUNK)],
        x_v.at[pl.ds(off, GCHUNK)], semx.at[j]))

  gcopies = []
  for j in range(NCHUNK):
    xcopies[j].wait()
    for g in range(GROUPS_PER_CHUNK):
      row0 = j * GCHUNK + g * LANES
      rows = row0 + lax.iota(jnp.int32, LANES)
      bits = [plsc.load_gather(x_v, [rows, jnp.full((LANES,), k, jnp.int32)])
              for k in range(NUM_INPUTS)]
      acc = _tree_sum([b * _W[k] for k, b in enumerate(bits)])
      idx_v[pl.ds(row0, LANES)] = acc.astype(jnp.int32)
    gcopies.append(pltpu.async_copy(
        lut_hbm.at[idx_v.at[pl.ds(j * GCHUNK, GCHUNK)]],
        vals_v.at[pl.ds(j * GCHUNK, GCHUNK)], semg.at[j]))

  ocopies = []
  for j in range(NCHUNK):
    gcopies[j].wait()
    for g in range(GROUPS_PER_CHUNK):
      off = j * GCHUNK + g * LANES
      gval = vals_v[pl.ds(off, LANES)]
      p = 1.0 / (1.0 + jnp.exp(-gval))
      out_v[pl.ds(off, LANES)] = jnp.where(
          p >= 0.5, jnp.float32(1.0), jnp.float32(0.0))
      probs_v[pl.ds(off, LANES)] = p
      logits_v[pl.ds(off, LANES)] = 5.0 * jnp.clip(gval, _LOGIT_LO, _LOGIT_HI)
    src = pl.ds(j * GCHUNK, GCHUNK)
    dst = pl.ds(base + j * GCHUNK, GCHUNK)
    ocopies.append(pltpu.async_copy(out_v.at[src], out_hbm.at[dst],
                                    semo.at[3 * j]))
    ocopies.append(pltpu.async_copy(probs_v.at[src], probs_hbm.at[dst],
                                    semo.at[3 * j + 1]))
    ocopies.append(pltpu.async_copy(logits_v.at[src], logits_hbm.at[dst],
                                    semo.at[3 * j + 2]))
  for c in ocopies:
    c.wait()


_OUT = jax.ShapeDtypeStruct((BATCH,), jnp.float32)

_logic_unit_sc = functools.partial(
    pl.kernel,
    out_type=(_OUT, _OUT, _OUT),
    mesh=plsc.VectorSubcoreMesh(core_axis_name="c", subcore_axis_name="s"),
    compiler_params=pltpu.CompilerParams(needs_layout_passes=False),
    scratch_types=[
        pltpu.VMEM((B_PER_W, NUM_INPUTS), jnp.float32),
        pltpu.VMEM((B_PER_W,), jnp.int32),
        pltpu.VMEM((B_PER_W,), jnp.float32),
        pltpu.VMEM((B_PER_W,), jnp.float32),
        pltpu.VMEM((B_PER_W,), jnp.float32),
        pltpu.VMEM((B_PER_W,), jnp.float32),
        pltpu.SemaphoreType.DMA((NCHUNK,)),
        pltpu.SemaphoreType.DMA((NCHUNK,)),
        pltpu.SemaphoreType.DMA((3 * NCHUNK,)),
    ],
)(_logic_unit_body)


@jax.jit
def kernel(x, lut_params):
  return _logic_unit_sc(x, lut_params)
